# Initial kernel scaffold; baseline (speedup 1.0000x reference)
import functools

import jax
import jax.numpy as jnp
from jax import lax
from jax.experimental import pallas as pl
from jax.experimental.pallas import tpu as pltpu
from jax.experimental.pallas import tpu_sc as plsc

_NT = 16
_CH = 128


def _make_sc_chain(NP, EP, F, K):
    NCH = EP // (_NT * _CH)
    RT = NP // _NT
    SB = RT // _CH
    FJ = F // 16
    assert K % 2 == 0

    mesh = plsc.VectorSubcoreMesh(core_axis_name="c", subcore_axis_name="s")

    @functools.partial(
        pl.kernel,
        out_type=jax.ShapeDtypeStruct((2, NP, F), jnp.float32),
        mesh=mesh,
        scratch_types=[
            pltpu.VMEM_SHARED((NP, F), jnp.float32),
            pltpu.VMEM_SHARED((NP, F), jnp.float32),
            pltpu.VMEM((NP // _NT, F), jnp.float32),
            pltpu.VMEM((_CH, F), jnp.float32),
            pltpu.VMEM((_CH, F), jnp.float32),
            pltpu.VMEM((_CH,), jnp.int32),
            pltpu.VMEM((_CH,), jnp.int32),
            pltpu.SMEM((_CH,), jnp.float32),
            pltpu.SMEM((K + 1,), jnp.float32),
        ],
    )
    def chain(xx, rows, cols, ew, fw, out, bufA, bufB, acc, gbuf, zbuf,
              colv, rowv, wsm, fwsm):
        c = lax.axis_index("c")
        s = lax.axis_index("s")
        r0 = s * RT

        pltpu.sync_copy(fw.at[c], fwsm)

        def zrow(r, _):
            for j in range(FJ):
                zbuf[r, pl.ds(16 * j, 16)] = jnp.zeros((16,), jnp.float32)
            return 0
        lax.fori_loop(0, _CH, zrow, 0)

        fw0 = fwsm[0]
        for sb in range(SB):
            rr = r0 + sb * _CH
            pltpu.sync_copy(xx.at[c, pl.ds(rr, _CH)], gbuf)
            pltpu.sync_copy(gbuf, bufA.at[pl.ds(rr, _CH)])

            def irow(r, _):
                for j in range(FJ):
                    dsj = pl.ds(16 * j, 16)
                    acc[sb * _CH + r, dsj] = gbuf[r, dsj] * fw0
                return 0
            lax.fori_loop(0, _CH, irow, 0)
            pltpu.sync_copy(zbuf, bufB.at[pl.ds(rr, _CH)])
        plsc.subcore_barrier()

        def hop(S, D, kidx):
            def chunk(j, _):
                base = (s * NCH + j) * _CH
                pltpu.sync_copy(cols.at[c, pl.ds(base, _CH)], colv)
                pltpu.sync_copy(rows.at[c, pl.ds(base, _CH)], rowv)
                pltpu.sync_copy(ew.at[c, pl.ds(base, _CH)], wsm)
                pltpu.sync_copy(S.at[colv], gbuf)

                def edge(e, _):
                    wv = wsm[e]
                    for j in range(FJ):
                        dsj = pl.ds(16 * j, 16)
                        gbuf[e, dsj] = gbuf[e, dsj] * wv
                    return 0
                lax.fori_loop(0, _CH, edge, 0)
                pltpu.sync_copy(gbuf, D.at[rowv], add=True)
                return 0
            lax.fori_loop(0, NCH, chunk, 0)
            plsc.subcore_barrier()

            coef = fwsm[kidx + 1]
            for sb in range(SB):
                rr = r0 + sb * _CH
                pltpu.sync_copy(D.at[pl.ds(rr, _CH)], gbuf)

                def arow(r, _):
                    for j in range(FJ):
                        dsj = pl.ds(16 * j, 16)
                        a = acc[sb * _CH + r, dsj]
                        acc[sb * _CH + r, dsj] = a + coef * gbuf[r, dsj]
                    return 0
                lax.fori_loop(0, _CH, arow, 0)
                pltpu.sync_copy(zbuf, S.at[pl.ds(rr, _CH)])
            plsc.subcore_barrier()

        def kkbody(kk, _):
            hop(bufA, bufB, 2 * kk)
            hop(bufB, bufA, 2 * kk + 1)
            return 0
        lax.fori_loop(0, K // 2, kkbody, 0)

        pltpu.sync_copy(acc, out.at[c, pl.ds(r0, RT)])

    return chain


def _ceil_to(v, m):
    return ((v + m - 1) // m) * m


def kernel(x, hl1_index, hl1_weight, hl2_index, hl2_weight, batch, params):
    N, Din = x.shape
    E = hl1_index.shape[1]
    NP = _ceil_to(N, _NT * _CH)
    EP = _ceil_to(E, _NT * _CH)
    F = params['layers'][0]['lin_in_w'][0].shape[0]
    K = params['layers'][0]['fW'][0].shape[0] - 1

    def pad_e(a):
        return jnp.pad(a, (0, EP - E))

    rows = jnp.stack([pad_e(hl1_index[0].astype(jnp.int32)),
                      pad_e(hl2_index[0].astype(jnp.int32))])
    cols = jnp.stack([pad_e(hl1_index[1].astype(jnp.int32)),
                      pad_e(hl2_index[1].astype(jnp.int32))])
    ew = jnp.stack([pad_e(hl1_weight), pad_e(hl2_weight)])

    chain = _make_sc_chain(NP, EP, F, K)

    eps = 1e-05
    h = x
    for lp in params['layers']:
        xx0 = h @ lp['lin_in_w'][0].T + lp['lin_in_b'][0]
        xx1 = h @ lp['lin_in_w'][1].T + lp['lin_in_b'][1]
        xxp = jnp.stack([jnp.pad(xx0, ((0, NP - N), (0, 0))),
                         jnp.pad(xx1, ((0, NP - N), (0, 0)))])
        fw = jnp.stack([lp['fW'][0], lp['fW'][1]])
        acc = chain(xxp, rows, cols, ew, fw)
        xc = jnp.concatenate([acc[0, :N], acc[1, :N]], axis=1)
        xc = xc @ lp['lin_out_w'].T + lp['lin_out_b']
        y = xc @ lp['nn_w1'].T + lp['nn_b1']
        y = jax.nn.relu(y / jnp.sqrt(1.0 + eps) * lp['bn1_g'] + lp['bn1_b'])
        y = y @ lp['nn_w2'].T + lp['nn_b2']
        y = jax.nn.relu(y / jnp.sqrt(1.0 + eps) * lp['bn2_g'] + lp['bn2_b'])
        h = y
    pooled = jax.ops.segment_sum(h, batch, num_segments=32)
    o = jax.nn.relu(pooled @ params['lin1_w'].T + params['lin1_b'])
    o = o @ params['lin2_w'].T + params['lin2_b']
    return o


# speedup vs baseline: 2.1494x; 2.1494x over previous
import functools

import jax
import jax.numpy as jnp
from jax import lax
from jax.experimental import pallas as pl
from jax.experimental.pallas import tpu as pltpu
from jax.experimental.pallas import tpu_sc as plsc

_NT = 16
_CH = 64


def _make_sc_chain(NP, EP, F, K):
    NCH = EP // (_NT * _CH)
    RT = NP // _NT
    SB = RT // _CH
    FJ = F // 16
    assert K % 2 == 0

    mesh = plsc.VectorSubcoreMesh(core_axis_name="c", subcore_axis_name="s")

    @functools.partial(
        pl.kernel,
        out_type=jax.ShapeDtypeStruct((2, NP, F), jnp.float32),
        mesh=mesh,
        compiler_params=pltpu.CompilerParams(use_tc_tiling_on_sc=False),
        scratch_types=[
            pltpu.VMEM_SHARED((NP, F), jnp.float32),
            pltpu.VMEM_SHARED((NP, F), jnp.float32),
            pltpu.VMEM((NP // _NT, F), jnp.float32),
            pltpu.VMEM((_CH, F), jnp.float32),
            pltpu.VMEM((_CH,), jnp.int32),
            pltpu.VMEM((_CH,), jnp.int32),
            pltpu.VMEM((_CH + 16,), jnp.float32),
            pltpu.VMEM((K + 17,), jnp.float32),
        ],
    )
    def chain(xx, rows, cols, ew, fw, out, bufA, bufB, acc, gbuf,
              colv, rowv, wvm, fwvm):
        c = lax.axis_index("c")
        s = lax.axis_index("s")
        r0 = s * RT

        pltpu.sync_copy(fw.at[c], fwvm.at[pl.ds(0, K + 1)])

        def zero_gbuf():
            def zrow(r, _):
                for j in range(FJ):
                    gbuf[r, pl.ds(16 * j, 16)] = jnp.zeros((16,), jnp.float32)
                return 0
            lax.fori_loop(0, _CH, zrow, 0)

        fw0 = fwvm[pl.ds(0, 16)][0]
        for sb in range(SB):
            rr = r0 + sb * _CH
            pltpu.sync_copy(xx.at[c, pl.ds(rr, _CH)], gbuf)
            pltpu.sync_copy(gbuf, bufA.at[pl.ds(rr, _CH)])

            def irow(r, _):
                for j in range(FJ):
                    dsj = pl.ds(16 * j, 16)
                    acc[sb * _CH + r, dsj] = gbuf[r, dsj] * fw0
                return 0
            lax.fori_loop(0, _CH, irow, 0)
        zero_gbuf()
        for sb in range(SB):
            rr = r0 + sb * _CH
            pltpu.sync_copy(gbuf, bufB.at[pl.ds(rr, _CH)])
        plsc.subcore_barrier()

        def hop(S, D, kidx):
            def chunk(j, _):
                base = (s * NCH + j) * _CH
                pltpu.sync_copy(cols.at[c, pl.ds(base, _CH)], colv)
                pltpu.sync_copy(rows.at[c, pl.ds(base, _CH)], rowv)
                pltpu.sync_copy(ew.at[c, pl.ds(base, _CH)],
                                wvm.at[pl.ds(0, _CH)])
                pltpu.sync_copy(S.at[colv], gbuf)

                def edge(e, _):
                    wv = wvm[pl.ds(e, 16)][0]
                    for j in range(FJ):
                        dsj = pl.ds(16 * j, 16)
                        gbuf[e, dsj] = gbuf[e, dsj] * wv
                    return 0
                lax.fori_loop(0, _CH, edge, 0)
                pltpu.sync_copy(gbuf, D.at[rowv], add=True)
                return 0
            lax.fori_loop(0, NCH, chunk, 0)
            plsc.subcore_barrier()

            coef = fwvm[pl.ds(kidx + 1, 16)][0]
            for sb in range(SB):
                rr = r0 + sb * _CH
                pltpu.sync_copy(D.at[pl.ds(rr, _CH)], gbuf)

                def arow(r, _):
                    for j in range(FJ):
                        dsj = pl.ds(16 * j, 16)
                        a = acc[sb * _CH + r, dsj]
                        acc[sb * _CH + r, dsj] = a + coef * gbuf[r, dsj]
                    return 0
                lax.fori_loop(0, _CH, arow, 0)
                zero_gbuf()
                pltpu.sync_copy(gbuf, S.at[pl.ds(rr, _CH)])
            plsc.subcore_barrier()

        def kkbody(kk, _):
            hop(bufA, bufB, 2 * kk)
            hop(bufB, bufA, 2 * kk + 1)
            return 0
        lax.fori_loop(0, K // 2, kkbody, 0)

        pltpu.sync_copy(acc, out.at[c, pl.ds(r0, RT)])

    return chain


def _ceil_to(v, m):
    return ((v + m - 1) // m) * m


def kernel(x, hl1_index, hl1_weight, hl2_index, hl2_weight, batch, params):
    N, Din = x.shape
    E = hl1_index.shape[1]
    NP = _ceil_to(N, _NT * _CH)
    EP = _ceil_to(E, _NT * _CH)
    F = params['layers'][0]['lin_in_w'][0].shape[0]
    K = params['layers'][0]['fW'][0].shape[0] - 1

    def pad_e(a):
        return jnp.pad(a, (0, EP - E))

    rows = jnp.stack([pad_e(hl1_index[0].astype(jnp.int32)),
                      pad_e(hl2_index[0].astype(jnp.int32))])
    cols = jnp.stack([pad_e(hl1_index[1].astype(jnp.int32)),
                      pad_e(hl2_index[1].astype(jnp.int32))])
    ew = jnp.stack([pad_e(hl1_weight), pad_e(hl2_weight)])

    chain = _make_sc_chain(NP, EP, F, K)

    eps = 1e-05
    h = x
    for lp in params['layers']:
        xx0 = h @ lp['lin_in_w'][0].T + lp['lin_in_b'][0]
        xx1 = h @ lp['lin_in_w'][1].T + lp['lin_in_b'][1]
        xxp = jnp.stack([jnp.pad(xx0, ((0, NP - N), (0, 0))),
                         jnp.pad(xx1, ((0, NP - N), (0, 0)))])
        fw = jnp.stack([lp['fW'][0], lp['fW'][1]])
        acc = chain(xxp, rows, cols, ew, fw)
        xc = jnp.concatenate([acc[0, :N], acc[1, :N]], axis=1)
        xc = xc @ lp['lin_out_w'].T + lp['lin_out_b']
        y = xc @ lp['nn_w1'].T + lp['nn_b1']
        y = jax.nn.relu(y / jnp.sqrt(1.0 + eps) * lp['bn1_g'] + lp['bn1_b'])
        y = y @ lp['nn_w2'].T + lp['nn_b2']
        y = jax.nn.relu(y / jnp.sqrt(1.0 + eps) * lp['bn2_g'] + lp['bn2_b'])
        h = y
    pooled = jax.ops.segment_sum(h, batch, num_segments=32)
    o = jax.nn.relu(pooled @ params['lin1_w'].T + params['lin1_b'])
    o = o @ params['lin2_w'].T + params['lin2_b']
    return o


# speedup vs baseline: 3.4384x; 1.5997x over previous
import functools

import jax
import jax.numpy as jnp
from jax import lax
from jax.experimental import pallas as pl
from jax.experimental.pallas import tpu as pltpu
from jax.experimental.pallas import tpu_sc as plsc

_NT = 16
_CH = 64
_NP = 10112


def _make_sc_chain(NP, NCH, F, K):
    RT = NP // _NT
    SBF = RT // _CH
    TAIL = RT - SBF * _CH
    FJ = F // 16
    assert K % 2 == 0 and NCH % 4 == 0

    mesh = plsc.VectorSubcoreMesh(core_axis_name="c", subcore_axis_name="s")

    @functools.partial(
        pl.kernel,
        out_type=jax.ShapeDtypeStruct((2, NP, F), jnp.float32),
        mesh=mesh,
        compiler_params=pltpu.CompilerParams(use_tc_tiling_on_sc=False,
                                             needs_layout_passes=False),
        scratch_types=[
            pltpu.VMEM_SHARED((NP, F), jnp.float32),
            pltpu.VMEM_SHARED((NP, F), jnp.float32),
            pltpu.VMEM((NP // _NT, F), jnp.float32),
            pltpu.VMEM((_CH, F), jnp.float32),
            pltpu.VMEM((_CH, F), jnp.float32),
            pltpu.VMEM((4, _CH), jnp.int32),
            pltpu.VMEM((4, _CH), jnp.int32),
            pltpu.VMEM((4, _CH), jnp.int32),
            pltpu.VMEM((4, _CH), jnp.int32),
            pltpu.VMEM((K + 17,), jnp.float32),
            pltpu.SemaphoreType.DMA,
            pltpu.SemaphoreType.DMA,
            pltpu.SemaphoreType.DMA,
            pltpu.SemaphoreType.DMA,
            pltpu.SemaphoreType.DMA,
            pltpu.SemaphoreType.DMA,
            pltpu.SemaphoreType.DMA,
            pltpu.SemaphoreType.DMA,
        ],
    )
    def chain(xx, packed, fw, out, bufA, bufB, acc, gbuf0, gbuf1,
              eb0, eb1, eb2, eb3, fwvm,
              esem0, esem1, esem2, esem3, gsem0, gsem1, ssem0, ssem1):
        c = lax.axis_index("c")
        s = lax.axis_index("s")
        r0 = s * RT
        ebs = (eb0, eb1, eb2, eb3)
        esems = (esem0, esem1, esem2, esem3)
        gbufs = (gbuf0, gbuf1)
        gsems = (gsem0, gsem1)
        ssems = (ssem0, ssem1)

        pltpu.sync_copy(fw.at[c], fwvm.at[pl.ds(0, K + 1)])

        def zero_gbuf(gb):
            def zrow(r, _):
                for j in range(FJ):
                    gb[r, pl.ds(16 * j, 16)] = jnp.zeros((16,), jnp.float32)
                return 0
            lax.fori_loop(0, _CH, zrow, 0)

        def for_each_subblock(fn):
            def body(sb, _):
                fn(sb * _CH, _CH)
                return 0
            lax.fori_loop(0, SBF, body, 0)
            if TAIL:
                fn(SBF * _CH, TAIL)

        fw0 = fwvm[pl.ds(0, 16)][0]

        def init_sb(ro, nr):
            pltpu.sync_copy(xx.at[c, pl.ds(r0 + ro, nr)],
                            gbuf0.at[pl.ds(0, nr)])
            pltpu.sync_copy(gbuf0.at[pl.ds(0, nr)],
                            bufA.at[pl.ds(r0 + ro, nr)])

            def irow(r, _):
                for j in range(FJ):
                    dsj = pl.ds(16 * j, 16)
                    acc[ro + r, dsj] = gbuf0[r, dsj] * fw0
                return 0
            lax.fori_loop(0, nr, irow, 0)
        for_each_subblock(init_sb)

        zero_gbuf(gbuf0)

        def zinit_sb(ro, nr):
            pltpu.sync_copy(gbuf0.at[pl.ds(0, nr)],
                            bufB.at[pl.ds(r0 + ro, nr)])
        for_each_subblock(zinit_sb)
        plsc.subcore_barrier()

        def edge_dma(q, j):
            return pltpu.async_copy(packed.at[c, s * NCH + j], ebs[q],
                                    esems[q])

        def hop(S, D, kidx):
            def g_issue(p, q):
                return pltpu.async_copy(S.at[ebs[q].at[0]], gbufs[p],
                                        gsems[p])

            def g_wait(p, q):
                pltpu.make_async_copy(S.at[ebs[q].at[0]], gbufs[p],
                                      gsems[p]).wait()

            def s_issue(p, q):
                return pltpu.async_copy(gbufs[p], D.at[ebs[q].at[1]],
                                        ssems[p], add=True)

            def s_wait(p, q):
                pltpu.make_async_copy(gbufs[p], D.at[ebs[q].at[1]],
                                      ssems[p]).wait()

            def multiply(p, q):
                gb = gbufs[p]
                eb = ebs[q]

                def grp(g, _):
                    wv16 = plsc.bitcast(eb[2, pl.ds(g * 16, 16)],
                                        jnp.float32)
                    for l in range(16):
                        wv = wv16[l]
                        e = g * 16 + l
                        for j in range(FJ):
                            dsj = pl.ds(16 * j, 16)
                            gb[e, dsj] = gb[e, dsj] * wv
                    return 0
                lax.fori_loop(0, _CH // 16, grp, 0)

            edge_dma(0, 0)
            edge_dma(1, 1)
            edge_dma(2, 2)
            pltpu.make_async_copy(packed.at[c, s * NCH], ebs[0],
                                  esems[0]).wait()
            g_issue(0, 0)

            def quad(jj, _):
                for u in range(4):
                    j = 4 * jj + u
                    p = u & 1
                    np_ = 1 - p
                    q = u
                    qn = (u + 1) % 4
                    qe = (u + 3) % 4
                    g_wait(p, q)
                    multiply(p, q)
                    if u == 0:
                        @pl.when(jj > 0)
                        def _():
                            s_wait(np_, qe)
                    else:
                        s_wait(np_, qe)
                    @pl.when(j + 3 < NCH)
                    def _():
                        edge_dma(qe, j + 3)
                    s_issue(p, q)
                    @pl.when(j + 1 < NCH)
                    def _():
                        pltpu.make_async_copy(packed.at[c, s * NCH + j + 1],
                                              ebs[qn], esems[qn]).wait()
                        g_issue(np_, qn)
                return 0
            lax.fori_loop(0, NCH // 4, quad, 0)
            s_wait(1, 3)
            plsc.subcore_barrier()

            coef = fwvm[pl.ds(kidx + 1, 16)][0]

            def axpy_sb(ro, nr):
                pltpu.sync_copy(D.at[pl.ds(r0 + ro, nr)],
                                gbuf0.at[pl.ds(0, nr)])

                def arow(r, _):
                    for j in range(FJ):
                        dsj = pl.ds(16 * j, 16)
                        a = acc[ro + r, dsj]
                        acc[ro + r, dsj] = a + coef * gbuf0[r, dsj]
                    return 0
                lax.fori_loop(0, nr, arow, 0)
                zero_gbuf(gbuf0)
                pltpu.sync_copy(gbuf0.at[pl.ds(0, nr)],
                                S.at[pl.ds(r0 + ro, nr)])
            for_each_subblock(axpy_sb)
            plsc.subcore_barrier()

        def kkbody(kk, _):
            hop(bufA, bufB, 2 * kk)
            hop(bufB, bufA, 2 * kk + 1)
            return 0
        lax.fori_loop(0, K // 2, kkbody, 0)

        pltpu.sync_copy(acc, out.at[c, pl.ds(r0, RT)])

    return chain


def _ceil_to(v, m):
    return ((v + m - 1) // m) * m


def kernel(x, hl1_index, hl1_weight, hl2_index, hl2_weight, batch, params):
    N, Din = x.shape
    E = hl1_index.shape[1]
    NP = _NP
    assert N <= NP
    NCH = _ceil_to(_ceil_to(E, _NT * _CH) // (_NT * _CH), 4)
    EP = NCH * _NT * _CH
    F = params['layers'][0]['lin_in_w'][0].shape[0]
    K = params['layers'][0]['fW'][0].shape[0] - 1

    def pack(idx, w):
        cols = jnp.pad(idx[1].astype(jnp.int32), (0, EP - E))
        rows = jnp.pad(idx[0].astype(jnp.int32), (0, EP - E))
        wb = jax.lax.bitcast_convert_type(jnp.pad(w, (0, EP - E)), jnp.int32)
        per = [a.reshape(_NT * NCH, _CH) for a in (cols, rows, wb, wb)]
        return jnp.stack(per, axis=1)

    packed = jnp.stack([pack(hl1_index, hl1_weight),
                        pack(hl2_index, hl2_weight)])

    chain = _make_sc_chain(NP, NCH, F, K)

    eps = 1e-05
    h = x
    for lp in params['layers']:
        xx0 = h @ lp['lin_in_w'][0].T + lp['lin_in_b'][0]
        xx1 = h @ lp['lin_in_w'][1].T + lp['lin_in_b'][1]
        xxp = jnp.stack([jnp.pad(xx0, ((0, NP - N), (0, 0))),
                         jnp.pad(xx1, ((0, NP - N), (0, 0)))])
        fw = jnp.stack([lp['fW'][0], lp['fW'][1]])
        acc = chain(xxp, packed, fw)
        xc = jnp.concatenate([acc[0, :N], acc[1, :N]], axis=1)
        xc = xc @ lp['lin_out_w'].T + lp['lin_out_b']
        y = xc @ lp['nn_w1'].T + lp['nn_b1']
        y = jax.nn.relu(y / jnp.sqrt(1.0 + eps) * lp['bn1_g'] + lp['bn1_b'])
        y = y @ lp['nn_w2'].T + lp['nn_b2']
        y = jax.nn.relu(y / jnp.sqrt(1.0 + eps) * lp['bn2_g'] + lp['bn2_b'])
        h = y
    pooled = jax.ops.segment_sum(h, batch, num_segments=32)
    o = jax.nn.relu(pooled @ params['lin1_w'].T + params['lin1_b'])
    o = o @ params['lin2_w'].T + params['lin2_b']
    return o


# speedup vs baseline: 10.4366x; 3.0353x over previous
import functools

import jax
import jax.numpy as jnp
from jax import lax
from jax.experimental import pallas as pl
from jax.experimental.pallas import tpu as pltpu
from jax.experimental.pallas import tpu_sc as plsc

_NT = 16
_CH = 128
_NP = 10112


def _make_sc_chain(NP, NCH, F, K):
    RT = NP // _NT
    SBF = RT // _CH
    TAIL = RT - SBF * _CH
    FJ = F // 16
    assert K % 2 == 0 and NCH % 8 == 0

    mesh = plsc.VectorSubcoreMesh(core_axis_name="c", subcore_axis_name="s")

    @functools.partial(
        pl.kernel,
        out_type=jax.ShapeDtypeStruct((2, K, NP, F), jnp.float32),
        mesh=mesh,
        compiler_params=pltpu.CompilerParams(use_tc_tiling_on_sc=False,
                                             needs_layout_passes=False),
        scratch_types=[
            pltpu.VMEM_SHARED((NP, F), jnp.float32),
            pltpu.VMEM_SHARED((NP, F), jnp.float32),
            *[pltpu.VMEM((_CH, F), jnp.float32) for _ in range(4)],
            *[pltpu.VMEM((4, _CH), jnp.int32) for _ in range(8)],
            *[pltpu.SemaphoreType.DMA for _ in range(16)],
            pltpu.SemaphoreType.DMA,
            pltpu.SemaphoreType.DMA,
        ],
    )
    def chain(xx, packed, zeros, out, bufA, bufB,
              gb0, gb1, gb2, gb3, e0, e1, e2, e3, e4, e5, e6, e7,
              es0, es1, es2, es3, es4, es5, es6, es7,
              gs0, gs1, gs2, gs3, ss0, ss1, ss2, ss3, zsem, hsem):
        gbufs = (gb0, gb1, gb2, gb3)
        ebufs = (e0, e1, e2, e3, e4, e5, e6, e7)
        esems = (es0, es1, es2, es3, es4, es5, es6, es7)
        gsems = (gs0, gs1, gs2, gs3)
        ssems = (ss0, ss1, ss2, ss3)
        c = lax.axis_index("c")
        s = lax.axis_index("s")
        r0 = s * RT

        pltpu.sync_copy(xx.at[c, pl.ds(r0, RT)], bufA.at[pl.ds(r0, RT)])
        pltpu.sync_copy(zeros.at[pl.ds(r0, RT)], bufB.at[pl.ds(r0, RT)])
        plsc.subcore_barrier()

        def edge_dma(j, q8):
            return pltpu.async_copy(packed.at[c, s * NCH + j], ebufs[q8],
                                    esems[q8])

        def edge_wait(j, q8):
            pltpu.make_async_copy(packed.at[c, s * NCH + j], ebufs[q8],
                                  esems[q8]).wait()

        def hop(S, D, kidx):
            def g_issue(q4, q8):
                return pltpu.async_copy(S.at[ebufs[q8].at[0]], gbufs[q4],
                                        gsems[q4])

            def g_wait(q4, q8):
                pltpu.make_async_copy(S.at[ebufs[q8].at[0]], gbufs[q4],
                                      gsems[q4]).wait()

            def s_issue(q4, q8):
                return pltpu.async_copy(gbufs[q4], D.at[ebufs[q8].at[1]],
                                        ssems[q4], add=True)

            def s_wait(q4, q8):
                pltpu.make_async_copy(gbufs[q4], D.at[ebufs[q8].at[1]],
                                      ssems[q4]).wait()

            def multiply(q4, q8):
                gb = gbufs[q4]
                eb = ebufs[q8]

                @plsc.parallel_loop(0, _CH // 16)
                def grp(g):
                    wv16 = plsc.bitcast(eb[2, pl.ds(g * 16, 16)],
                                        jnp.float32)
                    for l in range(16):
                        wv = wv16[l]
                        e = g * 16 + l
                        for j in range(FJ):
                            dsj = pl.ds(16 * j, 16)
                            gb[e, dsj] = gb[e, dsj] * wv

            def oct_(jj, _):
                for u in range(8):
                    j = 8 * jj + u
                    q4 = u % 4
                    q8 = u
                    g_wait(q4, q8)
                    if u >= 2:
                        s_wait((u + 2) % 4, (u + 6) % 8)
                    else:
                        @pl.when(jj > 0)
                        def _():
                            s_wait((u + 2) % 4, (u + 6) % 8)
                    @pl.when(j + 2 < NCH)
                    def _():
                        edge_wait(j + 2, (u + 2) % 8)
                        g_issue((u + 2) % 4, (u + 2) % 8)
                    @pl.when(j + 4 < NCH)
                    def _():
                        edge_dma(j + 4, (u + 4) % 8)
                    multiply(q4, q8)
                    s_issue(q4, q8)
                return 0
            lax.fori_loop(0, NCH // 8, oct_, 0)
            s_wait(2, 6)
            s_wait(3, 7)
            plsc.subcore_barrier()

            zcp = pltpu.async_copy(zeros.at[pl.ds(r0, RT)],
                                   S.at[pl.ds(r0, RT)], zsem)
            exp = pltpu.async_copy(D.at[pl.ds(r0, RT)],
                                   out.at[c, kidx, pl.ds(r0, RT)], hsem)

            @pl.when(kidx < K - 1)
            def _():
                prologue(D)
            zcp.wait()
            exp.wait()
            plsc.subcore_barrier()

        def prologue(Sn):
            for q in range(4):
                edge_dma(q, q)
            edge_wait(0, 0)
            pltpu.async_copy(Sn.at[ebufs[0].at[0]], gbufs[0], gsems[0])
            edge_wait(1, 1)
            pltpu.async_copy(Sn.at[ebufs[1].at[0]], gbufs[1], gsems[1])

        prologue(bufA)

        def kkbody(kk, _):
            hop(bufA, bufB, 2 * kk)
            hop(bufB, bufA, 2 * kk + 1)
            return 0
        lax.fori_loop(0, K // 2, kkbody, 0)

    return chain


def _ceil_to(v, m):
    return ((v + m - 1) // m) * m


_BLK = 1000


def _tc_lin2(h, w0t, b0, w1t, b1):
    N, Din = h.shape
    F = w0t.shape[1]
    grid = N // _BLK

    def body(h_ref, w0_ref, b0_ref, w1_ref, b1_ref, o0_ref, o1_ref):
        hb = h_ref[...]
        o0_ref[...] = jnp.dot(hb, w0_ref[...],
                              preferred_element_type=jnp.float32) + b0_ref[...]
        o1_ref[...] = jnp.dot(hb, w1_ref[...],
                              preferred_element_type=jnp.float32) + b1_ref[...]

    out = pl.pallas_call(
        body,
        grid=(grid,),
        in_specs=[
            pl.BlockSpec((_BLK, Din), lambda i: (i, 0)),
            pl.BlockSpec((Din, F), lambda i: (0, 0)),
            pl.BlockSpec((1, F), lambda i: (0, 0)),
            pl.BlockSpec((Din, F), lambda i: (0, 0)),
            pl.BlockSpec((1, F), lambda i: (0, 0)),
        ],
        out_specs=[pl.BlockSpec((_BLK, F), lambda i: (i, 0)),
                   pl.BlockSpec((_BLK, F), lambda i: (i, 0))],
        out_shape=[jax.ShapeDtypeStruct((N, F), jnp.float32)] * 2,
    )(h, w0t, b0[None], w1t, b1[None])
    return out


def _tc_mlp(xxp, slabs, fw, N, lp, eps):
    _, K, NP, F = slabs.shape
    grid = N // _BLK
    woA = lp['lin_out_w'][:, :F].T
    woB = lp['lin_out_w'][:, F:].T
    s1 = lp['bn1_g'] / jnp.sqrt(1.0 + eps)
    s2 = lp['bn2_g'] / jnp.sqrt(1.0 + eps)

    def body(fw_ref, xx_ref, sl_ref, woA_ref, woB_ref, bo_ref, w1_ref,
             b1_ref, s1_ref, t1_ref, w2_ref, b2_ref, s2_ref, t2_ref, o_ref):
        accs = []
        for ci in range(2):
            a = xx_ref[ci] * fw_ref[ci, 0]
            for k in range(K):
                a = a + sl_ref[ci, k] * fw_ref[ci, k + 1]
            accs.append(a)
        xc = (jnp.dot(accs[0], woA_ref[...],
                      preferred_element_type=jnp.float32)
              + jnp.dot(accs[1], woB_ref[...],
                        preferred_element_type=jnp.float32) + bo_ref[...])
        y = jnp.dot(xc, w1_ref[...],
                    preferred_element_type=jnp.float32) + b1_ref[...]
        y = jnp.maximum(y * s1_ref[...] + t1_ref[...], 0.0)
        y = jnp.dot(y, w2_ref[...],
                    preferred_element_type=jnp.float32) + b2_ref[...]
        o_ref[...] = jnp.maximum(y * s2_ref[...] + t2_ref[...], 0.0)

    vec = lambda: pl.BlockSpec((1, F), lambda i: (0, 0))
    mat = lambda: pl.BlockSpec((F, F), lambda i: (0, 0))
    out = pl.pallas_call(
        body,
        grid=(grid,),
        in_specs=[
            pl.BlockSpec(memory_space=pltpu.SMEM),
            pl.BlockSpec((2, _BLK, F), lambda i: (0, i, 0)),
            pl.BlockSpec((2, K, _BLK, F), lambda i: (0, 0, i, 0)),
            mat(), mat(), vec(), mat(), vec(), vec(), vec(),
            mat(), vec(), vec(), vec(),
        ],
        out_specs=pl.BlockSpec((_BLK, F), lambda i: (i, 0)),
        out_shape=jax.ShapeDtypeStruct((N, F), jnp.float32),
    )(fw, xxp, slabs, woA, woB, lp['lin_out_b'][None], lp['nn_w1'].T,
      lp['nn_b1'][None], s1[None], lp['bn1_b'][None], lp['nn_w2'].T,
      lp['nn_b2'][None], s2[None], lp['bn2_b'][None])
    return out


def _tc_pool_head(h, batch, ngraph, l1t, b1, l2t, b2):
    N, F = h.shape
    NCLS = l2t.shape[1]
    grid = N // _BLK
    batch3 = batch.astype(jnp.int32).reshape(grid, 1, _BLK)

    def body(h_ref, b_ref, l1_ref, b1_ref, l2_ref, b2_ref, o_ref, pool_ref):
        i = pl.program_id(0)

        @pl.when(i == 0)
        def _():
            pool_ref[...] = jnp.zeros_like(pool_ref)

        seg = b_ref[0]
        gids = jax.lax.broadcasted_iota(jnp.int32, (ngraph, _BLK), 0)
        mask = (gids == seg).astype(jnp.float32)
        pool_ref[...] += jnp.dot(mask, h_ref[...],
                                 preferred_element_type=jnp.float32)

        @pl.when(i == grid - 1)
        def _():
            p = jnp.maximum(jnp.dot(pool_ref[...], l1_ref[...],
                                    preferred_element_type=jnp.float32)
                            + b1_ref[...], 0.0)
            o_ref[...] = jnp.dot(p, l2_ref[...],
                                 preferred_element_type=jnp.float32) + b2_ref[...]

    out = pl.pallas_call(
        body,
        grid=(grid,),
        in_specs=[
            pl.BlockSpec((_BLK, F), lambda i: (i, 0)),
            pl.BlockSpec((1, 1, _BLK), lambda i: (i, 0, 0)),
            pl.BlockSpec((F, F), lambda i: (0, 0)),
            pl.BlockSpec((1, F), lambda i: (0, 0)),
            pl.BlockSpec((F, NCLS), lambda i: (0, 0)),
            pl.BlockSpec((1, NCLS), lambda i: (0, 0)),
        ],
        out_specs=pl.BlockSpec((ngraph, NCLS), lambda i: (0, 0)),
        out_shape=jax.ShapeDtypeStruct((ngraph, NCLS), jnp.float32),
        scratch_shapes=[pltpu.VMEM((ngraph, F), jnp.float32)],
    )(h, batch3, l1t, b1[None], l2t, b2[None])
    return out


def kernel(x, hl1_index, hl1_weight, hl2_index, hl2_weight, batch, params):
    N, Din = x.shape
    E = hl1_index.shape[1]
    NP = _NP
    assert N <= NP
    NCH = _ceil_to(_ceil_to(E, _NT * _CH) // (_NT * _CH), 8)
    EP = NCH * _NT * _CH
    F = params['layers'][0]['lin_in_w'][0].shape[0]
    K = params['layers'][0]['fW'][0].shape[0] - 1

    def pack(idx, w):
        cols = jnp.pad(idx[1].astype(jnp.int32), (0, EP - E))
        rows = jnp.pad(idx[0].astype(jnp.int32), (0, EP - E))
        wb = jax.lax.bitcast_convert_type(jnp.pad(w, (0, EP - E)), jnp.int32)
        per = [a.reshape(_NT * NCH, _CH) for a in (cols, rows, wb, wb)]
        return jnp.stack(per, axis=1)

    packed = jnp.stack([pack(hl1_index, hl1_weight),
                        pack(hl2_index, hl2_weight)])

    chain = _make_sc_chain(NP, NCH, F, K)
    zeros = jnp.zeros((NP, F), jnp.float32)

    eps = 1e-05
    h = x
    for lp in params['layers']:
        xx0, xx1 = _tc_lin2(h, lp['lin_in_w'][0].T, lp['lin_in_b'][0],
                            lp['lin_in_w'][1].T, lp['lin_in_b'][1])
        xxp = jnp.stack([jnp.pad(xx0, ((0, NP - N), (0, 0))),
                         jnp.pad(xx1, ((0, NP - N), (0, 0)))])
        fw = jnp.stack([lp['fW'][0], lp['fW'][1]])
        slabs = chain(xxp, packed, zeros)
        h = _tc_mlp(xxp, slabs, fw, N, lp, eps)
    o = _tc_pool_head(h, batch, 32, params['lin1_w'].T, params['lin1_b'],
                      params['lin2_w'].T, params['lin2_b'])
    return o


# speedup vs baseline: 10.7724x; 1.0322x over previous
import functools

import jax
import jax.numpy as jnp
from jax import lax
from jax.experimental import pallas as pl
from jax.experimental.pallas import tpu as pltpu
from jax.experimental.pallas import tpu_sc as plsc

_NT = 16
_CH = 128
_NP = 10112


def _make_sc_chain(NP, NCH, F, K):
    RT = NP // _NT
    SBF = RT // _CH
    TAIL = RT - SBF * _CH
    FJ = F // 16
    assert K % 2 == 0 and NCH % 8 == 0

    mesh = plsc.VectorSubcoreMesh(core_axis_name="c", subcore_axis_name="s")

    @functools.partial(
        pl.kernel,
        out_type=jax.ShapeDtypeStruct((2, K, NP, F), jnp.float32),
        mesh=mesh,
        compiler_params=pltpu.CompilerParams(use_tc_tiling_on_sc=False,
                                             needs_layout_passes=False),
        scratch_types=[
            pltpu.VMEM_SHARED((NP, F), jnp.float32),
            pltpu.VMEM_SHARED((NP, F), jnp.float32),
            *[pltpu.VMEM((_CH, F), jnp.float32) for _ in range(4)],
            *[pltpu.VMEM((4, _CH), jnp.int32) for _ in range(8)],
            *[pltpu.SemaphoreType.DMA for _ in range(16)],
            pltpu.SemaphoreType.DMA,
            pltpu.SemaphoreType.DMA,
        ],
    )
    def chain(xx, packed, zeros, out, bufA, bufB,
              gb0, gb1, gb2, gb3, e0, e1, e2, e3, e4, e5, e6, e7,
              es0, es1, es2, es3, es4, es5, es6, es7,
              gs0, gs1, gs2, gs3, ss0, ss1, ss2, ss3, zsem, hsem):
        gbufs = (gb0, gb1, gb2, gb3)
        ebufs = (e0, e1, e2, e3, e4, e5, e6, e7)
        esems = (es0, es1, es2, es3, es4, es5, es6, es7)
        gsems = (gs0, gs1, gs2, gs3)
        ssems = (ss0, ss1, ss2, ss3)
        c = lax.axis_index("c")
        s = lax.axis_index("s")
        r0 = s * RT

        pltpu.sync_copy(xx.at[c, pl.ds(r0, RT)], bufA.at[pl.ds(r0, RT)])
        pltpu.sync_copy(zeros.at[pl.ds(r0, RT)], bufB.at[pl.ds(r0, RT)])
        plsc.subcore_barrier()

        def edge_dma(j, q8):
            return pltpu.async_copy(packed.at[c, s * NCH + j], ebufs[q8],
                                    esems[q8])

        def edge_wait(j, q8):
            pltpu.make_async_copy(packed.at[c, s * NCH + j], ebufs[q8],
                                  esems[q8]).wait()

        def hop(S, D, kidx):
            def g_issue(q4, q8):
                return pltpu.async_copy(S.at[ebufs[q8].at[0]], gbufs[q4],
                                        gsems[q4])

            def g_wait(q4, q8):
                pltpu.make_async_copy(S.at[ebufs[q8].at[0]], gbufs[q4],
                                      gsems[q4]).wait()

            def s_issue(q4, q8):
                return pltpu.async_copy(gbufs[q4], D.at[ebufs[q8].at[1]],
                                        ssems[q4], add=True)

            def s_wait(q4, q8):
                pltpu.make_async_copy(gbufs[q4], D.at[ebufs[q8].at[1]],
                                      ssems[q4]).wait()

            def multiply(q4, q8):
                gb = gbufs[q4]
                eb = ebufs[q8]

                def grp(g, _):
                    wv16 = plsc.bitcast(eb[2, pl.ds(g * 16, 16)],
                                        jnp.float32)
                    for l in range(16):
                        wv = wv16[l]
                        e = g * 16 + l
                        for j in range(FJ):
                            dsj = pl.ds(16 * j, 16)
                            gb[e, dsj] = gb[e, dsj] * wv
                    return 0
                lax.fori_loop(0, _CH // 16, grp, 0)

            def oct_(jj, _):
                for u in range(8):
                    j = 8 * jj + u
                    q4 = u % 4
                    q8 = u
                    g_wait(q4, q8)
                    if u >= 2:
                        s_wait((u + 2) % 4, (u + 6) % 8)
                    else:
                        @pl.when(jj > 0)
                        def _():
                            s_wait((u + 2) % 4, (u + 6) % 8)
                    @pl.when(j + 2 < NCH)
                    def _():
                        edge_wait(j + 2, (u + 2) % 8)
                        g_issue((u + 2) % 4, (u + 2) % 8)
                    @pl.when(j + 4 < NCH)
                    def _():
                        edge_dma(j + 4, (u + 4) % 8)
                    multiply(q4, q8)
                    s_issue(q4, q8)
                return 0
            lax.fori_loop(0, NCH // 8, oct_, 0)
            s_wait(2, 6)
            s_wait(3, 7)
            plsc.subcore_barrier()

            zcp = pltpu.async_copy(zeros.at[pl.ds(r0, RT)],
                                   S.at[pl.ds(r0, RT)], zsem)
            exp = pltpu.async_copy(D.at[pl.ds(r0, RT)],
                                   out.at[c, kidx, pl.ds(r0, RT)], hsem)

            @pl.when(kidx < K - 1)
            def _():
                prologue(D)
            zcp.wait()
            exp.wait()
            plsc.subcore_barrier()

        def prologue(Sn):
            for q in range(4):
                edge_dma(q, q)
            edge_wait(0, 0)
            pltpu.async_copy(Sn.at[ebufs[0].at[0]], gbufs[0], gsems[0])
            edge_wait(1, 1)
            pltpu.async_copy(Sn.at[ebufs[1].at[0]], gbufs[1], gsems[1])

        prologue(bufA)

        def kkbody(kk, _):
            hop(bufA, bufB, 2 * kk)
            hop(bufB, bufA, 2 * kk + 1)
            return 0
        lax.fori_loop(0, K // 2, kkbody, 0)

    return chain


def _ceil_to(v, m):
    return ((v + m - 1) // m) * m


_BLK = 1000


def _tc_lin2(h, w0t, b0, w1t, b1):
    N, Din = h.shape
    F = w0t.shape[1]
    grid = N // _BLK

    def body(h_ref, w0_ref, b0_ref, w1_ref, b1_ref, o0_ref, o1_ref):
        hb = h_ref[...]
        o0_ref[...] = jnp.dot(hb, w0_ref[...],
                              preferred_element_type=jnp.float32) + b0_ref[...]
        o1_ref[...] = jnp.dot(hb, w1_ref[...],
                              preferred_element_type=jnp.float32) + b1_ref[...]

    out = pl.pallas_call(
        body,
        grid=(grid,),
        in_specs=[
            pl.BlockSpec((_BLK, Din), lambda i: (i, 0)),
            pl.BlockSpec((Din, F), lambda i: (0, 0)),
            pl.BlockSpec((1, F), lambda i: (0, 0)),
            pl.BlockSpec((Din, F), lambda i: (0, 0)),
            pl.BlockSpec((1, F), lambda i: (0, 0)),
        ],
        out_specs=[pl.BlockSpec((_BLK, F), lambda i: (i, 0)),
                   pl.BlockSpec((_BLK, F), lambda i: (i, 0))],
        out_shape=[jax.ShapeDtypeStruct((N, F), jnp.float32)] * 2,
    )(h, w0t, b0[None], w1t, b1[None])
    return out


def _tc_mlp(xxp, slabs, fw, N, lp, eps):
    _, K, NP, F = slabs.shape
    grid = N // _BLK
    woA = lp['lin_out_w'][:, :F].T
    woB = lp['lin_out_w'][:, F:].T
    s1 = lp['bn1_g'] / jnp.sqrt(1.0 + eps)
    s2 = lp['bn2_g'] / jnp.sqrt(1.0 + eps)

    def body(fw_ref, xx_ref, sl_ref, woA_ref, woB_ref, bo_ref, w1_ref,
             b1_ref, s1_ref, t1_ref, w2_ref, b2_ref, s2_ref, t2_ref, o_ref):
        accs = []
        for ci in range(2):
            a = xx_ref[ci] * fw_ref[ci, 0]
            for k in range(K):
                a = a + sl_ref[ci, k] * fw_ref[ci, k + 1]
            accs.append(a)
        xc = (jnp.dot(accs[0], woA_ref[...],
                      preferred_element_type=jnp.float32)
              + jnp.dot(accs[1], woB_ref[...],
                        preferred_element_type=jnp.float32) + bo_ref[...])
        y = jnp.dot(xc, w1_ref[...],
                    preferred_element_type=jnp.float32) + b1_ref[...]
        y = jnp.maximum(y * s1_ref[...] + t1_ref[...], 0.0)
        y = jnp.dot(y, w2_ref[...],
                    preferred_element_type=jnp.float32) + b2_ref[...]
        o_ref[...] = jnp.maximum(y * s2_ref[...] + t2_ref[...], 0.0)

    vec = lambda: pl.BlockSpec((1, F), lambda i: (0, 0))
    mat = lambda: pl.BlockSpec((F, F), lambda i: (0, 0))
    out = pl.pallas_call(
        body,
        grid=(grid,),
        in_specs=[
            pl.BlockSpec(memory_space=pltpu.SMEM),
            pl.BlockSpec((2, _BLK, F), lambda i: (0, i, 0)),
            pl.BlockSpec((2, K, _BLK, F), lambda i: (0, 0, i, 0)),
            mat(), mat(), vec(), mat(), vec(), vec(), vec(),
            mat(), vec(), vec(), vec(),
        ],
        out_specs=pl.BlockSpec((_BLK, F), lambda i: (i, 0)),
        out_shape=jax.ShapeDtypeStruct((N, F), jnp.float32),
    )(fw, xxp, slabs, woA, woB, lp['lin_out_b'][None], lp['nn_w1'].T,
      lp['nn_b1'][None], s1[None], lp['bn1_b'][None], lp['nn_w2'].T,
      lp['nn_b2'][None], s2[None], lp['bn2_b'][None])
    return out


def _tc_pool_head(h, batch, ngraph, l1t, b1, l2t, b2):
    N, F = h.shape
    NCLS = l2t.shape[1]
    grid = N // _BLK
    batch3 = batch.astype(jnp.int32).reshape(grid, 1, _BLK)

    def body(h_ref, b_ref, l1_ref, b1_ref, l2_ref, b2_ref, o_ref, pool_ref):
        i = pl.program_id(0)

        @pl.when(i == 0)
        def _():
            pool_ref[...] = jnp.zeros_like(pool_ref)

        seg = b_ref[0]
        gids = jax.lax.broadcasted_iota(jnp.int32, (ngraph, _BLK), 0)
        mask = (gids == seg).astype(jnp.float32)
        pool_ref[...] += jnp.dot(mask, h_ref[...],
                                 preferred_element_type=jnp.float32)

        @pl.when(i == grid - 1)
        def _():
            p = jnp.maximum(jnp.dot(pool_ref[...], l1_ref[...],
                                    preferred_element_type=jnp.float32)
                            + b1_ref[...], 0.0)
            o_ref[...] = jnp.dot(p, l2_ref[...],
                                 preferred_element_type=jnp.float32) + b2_ref[...]

    out = pl.pallas_call(
        body,
        grid=(grid,),
        in_specs=[
            pl.BlockSpec((_BLK, F), lambda i: (i, 0)),
            pl.BlockSpec((1, 1, _BLK), lambda i: (i, 0, 0)),
            pl.BlockSpec((F, F), lambda i: (0, 0)),
            pl.BlockSpec((1, F), lambda i: (0, 0)),
            pl.BlockSpec((F, NCLS), lambda i: (0, 0)),
            pl.BlockSpec((1, NCLS), lambda i: (0, 0)),
        ],
        out_specs=pl.BlockSpec((ngraph, NCLS), lambda i: (0, 0)),
        out_shape=jax.ShapeDtypeStruct((ngraph, NCLS), jnp.float32),
        scratch_shapes=[pltpu.VMEM((ngraph, F), jnp.float32)],
    )(h, batch3, l1t, b1[None], l2t, b2[None])
    return out


def kernel(x, hl1_index, hl1_weight, hl2_index, hl2_weight, batch, params):
    N, Din = x.shape
    E = hl1_index.shape[1]
    NP = _NP
    assert N <= NP
    NCH = _ceil_to(_ceil_to(E, _NT * _CH) // (_NT * _CH), 8)
    EP = NCH * _NT * _CH
    F = params['layers'][0]['lin_in_w'][0].shape[0]
    K = params['layers'][0]['fW'][0].shape[0] - 1

    def pack(idx, w):
        cols = jnp.pad(idx[1].astype(jnp.int32), (0, EP - E))
        rows = jnp.pad(idx[0].astype(jnp.int32), (0, EP - E))
        wb = jax.lax.bitcast_convert_type(jnp.pad(w, (0, EP - E)), jnp.int32)
        per = [a.reshape(_NT * NCH, _CH) for a in (cols, rows, wb, wb)]
        return jnp.stack(per, axis=1)

    packed = jnp.stack([pack(hl1_index, hl1_weight),
                        pack(hl2_index, hl2_weight)])

    chain = _make_sc_chain(NP, NCH, F, K)
    zeros = jnp.zeros((NP, F), jnp.float32)

    eps = 1e-05
    h = x
    for lp in params['layers']:
        xx0, xx1 = _tc_lin2(h, lp['lin_in_w'][0].T, lp['lin_in_b'][0],
                            lp['lin_in_w'][1].T, lp['lin_in_b'][1])
        xxp = jnp.stack([jnp.pad(xx0, ((0, NP - N), (0, 0))),
                         jnp.pad(xx1, ((0, NP - N), (0, 0)))])
        fw = jnp.stack([lp['fW'][0], lp['fW'][1]])
        slabs = chain(xxp, packed, zeros)
        h = _tc_mlp(xxp, slabs, fw, N, lp, eps)
    o = _tc_pool_head(h, batch, 32, params['lin1_w'].T, params['lin1_b'],
                      params['lin2_w'].T, params['lin2_b'])
    return o
